# Initial kernel scaffold; baseline (speedup 1.0000x reference)
import jax
import jax.numpy as jnp
from jax.experimental import pallas as pl


def kernel(x, raw_weight):
    raise NotImplementedError("write your pallas kernel here")


# speedup vs baseline: 1.2616x; 1.2616x over previous
import functools

import jax
import jax.numpy as jnp
from jax import lax
from jax.experimental import pallas as pl
from jax.experimental.pallas import tpu as pltpu

IN_F = 1024
OUT_F = 1024
BATCH = 4096

def _onehot_body(w_ref, p_ref):
    w = w_ref[...]
    row_max = jnp.max(w, axis=1, keepdims=True)
    col = lax.broadcasted_iota(jnp.int32, w.shape, 1)
    masked = jnp.where(w == row_max, col, 2**30)
    idx = jnp.min(masked, axis=1, keepdims=True)
    p_ref[...] = (col == idx).astype(jnp.float32)


def _onehot(raw_weight):
    return pl.pallas_call(
        _onehot_body,
        out_shape=jax.ShapeDtypeStruct((OUT_F, IN_F), jnp.float32),
    )(raw_weight)


_BB = 512


def _gather_body(x_ref, p_ref, o_ref):
    o_ref[...] = lax.dot_general(
        x_ref[...], p_ref[...],
        (((1,), (1,)), ((), ())),
        preferred_element_type=jnp.float32,
        precision=lax.Precision.HIGHEST,
    )


def _gather(x, p_t):
    return pl.pallas_call(
        _gather_body,
        grid=(BATCH // _BB,),
        in_specs=[
            pl.BlockSpec((_BB, IN_F), lambda i: (i, 0)),
            pl.BlockSpec((OUT_F, IN_F), lambda i: (0, 0)),
        ],
        out_specs=pl.BlockSpec((_BB, OUT_F), lambda i: (i, 0)),
        out_shape=jax.ShapeDtypeStruct((BATCH, OUT_F), jnp.float32),
    )(x, p_t)


@jax.jit
def kernel(x, raw_weight):
    p_t = _onehot(raw_weight)
    return _gather(x, p_t)
